# Initial kernel scaffold; baseline (speedup 1.0000x reference)
import jax
import jax.numpy as jnp
from jax.experimental import pallas as pl


def kernel(x, idf):
    raise NotImplementedError("write your pallas kernel here")


# speedup vs baseline: 1.2388x; 1.2388x over previous
import functools

import jax
import jax.numpy as jnp
from jax import lax
from jax.experimental import pallas as pl
from jax.experimental.pallas import tpu as pltpu
from jax.experimental.pallas import tpu_sc as plsc

LANES = 16
NC = 2
NS = 16
NW = NC * NS


def _document_model(x, idf, B, L):
    V = idf.shape[0]
    rows_per = B // NW
    n_chunks = (L + LANES - 1) // LANES
    l_pad = n_chunks * LANES
    rem = L - (n_chunks - 1) * LANES
    v_pad = ((V + 127) // 128) * 128
    half = l_pad // 2

    mesh = plsc.VectorSubcoreMesh(core_axis_name="c", subcore_axis_name="s")

    dnums = lax.GatherDimensionNumbers(
        offset_dims=(), collapsed_slice_dims=(0,), start_index_map=(0,))

    def lane_perm(v, idx):
        return lax.gather(v, idx[:, None], dnums, (1,),
                          mode=lax.GatherScatterMode.PROMISE_IN_BOUNDS)

    def lane_sum(v, lane):
        for s in (8, 4, 2, 1):
            v = v + lane_perm(v, lane ^ s)
        return v

    @functools.partial(
        pl.kernel,
        mesh=mesh,
        out_type=jax.ShapeDtypeStruct((B, V), jnp.float32),
        compiler_params=pltpu.CompilerParams(
            needs_layout_passes=False, use_tc_tiling_on_sc=False),
        scratch_types=[
            pltpu.VMEM((l_pad,), jnp.int32),
            pltpu.VMEM((l_pad,), jnp.float32),
            pltpu.VMEM((v_pad,), jnp.float32),
        ],
    )
    def run(x_hbm, idf_hbm, out_hbm, tok_ref, vals_ref, row_ref):
        cid = lax.axis_index("c")
        sid = lax.axis_index("s")
        wid = sid * NC + cid

        zf = jnp.zeros((LANES,), jnp.float32)
        zi = jnp.zeros((LANES,), jnp.int32)
        lane = lax.iota(jnp.int32, LANES)

        for i in range(n_chunks):
            tok_ref[pl.ds(LANES * i, LANES)] = zi

        def zero_body(i, c):
            for j in range(8):
                row_ref[pl.ds(i * 128 + j * LANES, LANES)] = zf
            return c

        lax.fori_loop(0, v_pad // 128, zero_body, 0)

        def row_body(r, c):
            row = wid * rows_per + r
            pltpu.sync_copy(x_hbm.at[pl.ds(row * L, L)], tok_ref.at[pl.ds(0, L)])
            pltpu.sync_copy(idf_hbm.at[tok_ref.at[pl.ds(0, half)]],
                            vals_ref.at[pl.ds(0, half)])
            pltpu.sync_copy(idf_hbm.at[tok_ref.at[pl.ds(half, half)]],
                            vals_ref.at[pl.ds(half, half)])
            acc = zf
            for i in range(n_chunks):
                v = vals_ref[pl.ds(LANES * i, LANES)]
                if i == n_chunks - 1 and rem != LANES:
                    v = jnp.where(lane < rem, v, 0.0)
                acc = acc + v
            inv = 1.0 / lane_sum(acc, lane)
            for i in range(n_chunks):
                idx = tok_ref[pl.ds(LANES * i, LANES)]
                v = vals_ref[pl.ds(LANES * i, LANES)] * inv
                if i == n_chunks - 1 and rem != LANES:
                    v = jnp.where(lane < rem, v, 0.0)
                plsc.addupdate_scatter(row_ref, [idx], v)
            pltpu.sync_copy(row_ref.at[pl.ds(0, V)], out_hbm.at[row])
            for i in range(n_chunks):
                idx = tok_ref[pl.ds(LANES * i, LANES)]
                plsc.store_scatter(row_ref, [idx], zf)
            return c

        lax.fori_loop(0, rows_per, row_body, 0)

    return run(x, idf)


def kernel(x, idf):
    B, L = x.shape
    return _document_model(x.astype(jnp.int32).reshape(-1), idf, B, L)


# speedup vs baseline: 2.1536x; 1.7384x over previous
import functools

import jax
import jax.numpy as jnp
from jax import lax
from jax.experimental import pallas as pl
from jax.experimental.pallas import tpu as pltpu
from jax.experimental.pallas import tpu_sc as plsc

LANES = 16
NC = 2
NS = 16
NW = NC * NS
TILE_R, TILE_C = 8, 128


def _document_model(x_flat, idf, B, L):
    V = idf.shape[0]
    NT = V // TILE_C
    rem_c = V - NT * TILE_C
    t_s = max(d for d in range(1, NT + 1)
              if NT % d == 0 and TILE_R * TILE_C * d <= 110_000)
    W = t_s * TILE_C
    n_strips = NT // t_s
    rbw = B // (TILE_R * NW)
    n_chunks = (L + LANES - 1) // LANES
    l_pad = n_chunks * LANES
    rem_l = L - (n_chunks - 1) * LANES
    g_chunk = 104
    n_gather = (TILE_R * l_pad) // g_chunk

    mesh = plsc.VectorSubcoreMesh(core_axis_name="c", subcore_axis_name="s")

    dnums = lax.GatherDimensionNumbers(
        offset_dims=(), collapsed_slice_dims=(0,), start_index_map=(0,))

    def lane_perm(v, idx):
        return lax.gather(v, idx[:, None], dnums, (1,),
                          mode=lax.GatherScatterMode.PROMISE_IN_BOUNDS)

    def lane_sum(v, lane):
        for s in (8, 4, 2, 1):
            v = v + lane_perm(v, lane ^ s)
        return v

    @functools.partial(
        pl.kernel,
        mesh=mesh,
        out_type=jax.ShapeDtypeStruct((B, V), jnp.float32),
        compiler_params=pltpu.CompilerParams(needs_layout_passes=False),
        scratch_types=[
            pltpu.VMEM((TILE_R * l_pad,), jnp.int32),
            pltpu.VMEM((TILE_R * l_pad,), jnp.float32),
            pltpu.VMEM((TILE_R, W), jnp.float32),
            pltpu.VMEM((TILE_R, max(rem_c, 1)), jnp.float32),
            pltpu.SemaphoreType.DMA,
        ],
    )
    def run(x_hbm, idf_hbm, out_hbm, tok_ref, vals_ref, sbuf, pbuf, sem):
        wid = lax.axis_index("s") * NC + lax.axis_index("c")
        lane = lax.iota(jnp.int32, LANES)
        zf = jnp.zeros((LANES,), jnp.float32)
        zi = jnp.zeros((LANES,), jnp.int32)
        row_ids = [jnp.full((LANES,), r8, jnp.int32) for r8 in range(TILE_R)]

        for i in range(TILE_R * l_pad // LANES):
            tok_ref[pl.ds(LANES * i, LANES)] = zi
        if rem_c:
            for r8 in range(TILE_R):
                for i in range(rem_c // LANES):
                    pbuf[r8, pl.ds(LANES * i, LANES)] = zf

        def zero_body(i, c):
            for r8 in range(TILE_R):
                sbuf[r8, pl.ds(i * LANES, LANES)] = zf
            return c

        lax.fori_loop(0, W // LANES, zero_body, 0)

        def rb_body(k, c):
            r0 = (wid * rbw + k) * TILE_R
            hs = [pltpu.async_copy(x_hbm.at[pl.ds((r0 + r8) * L, L)],
                                   tok_ref.at[pl.ds(r8 * l_pad, L)], sem)
                  for r8 in range(TILE_R)]
            for h in hs:
                h.wait()
            hs = [pltpu.async_copy(
                      idf_hbm.at[tok_ref.at[pl.ds(g_chunk * j, g_chunk)]],
                      vals_ref.at[pl.ds(g_chunk * j, g_chunk)], sem)
                  for j in range(n_gather)]
            for h in hs:
                h.wait()
            for r8 in range(TILE_R):
                base = r8 * l_pad
                acc = zf
                for i in range(n_chunks):
                    v = vals_ref[pl.ds(base + LANES * i, LANES)]
                    if i == n_chunks - 1 and rem_l != LANES:
                        v = jnp.where(lane < rem_l, v, 0.0)
                    acc = acc + v
                inv = 1.0 / lane_sum(acc, lane)
                for i in range(n_chunks):
                    v = vals_ref[pl.ds(base + LANES * i, LANES)] * inv
                    if i == n_chunks - 1 and rem_l != LANES:
                        v = jnp.where(lane < rem_l, v, 0.0)
                    vals_ref[pl.ds(base + LANES * i, LANES)] = v
            def strip_body(s, c2):
                lo = s * W
                for r8 in range(TILE_R):
                    base = r8 * l_pad
                    for i in range(n_chunks):
                        t = tok_ref[pl.ds(base + LANES * i, LANES)] - lo
                        val = vals_ref[pl.ds(base + LANES * i, LANES)]
                        m = (t >= 0) & (t < W)
                        plsc.addupdate_scatter(sbuf, [row_ids[r8], t], val,
                                               mask=m)
                pltpu.sync_copy(sbuf, out_hbm.at[pl.ds(r0, TILE_R),
                                                 pl.ds(lo, W)])
                for r8 in range(TILE_R):
                    base = r8 * l_pad
                    for i in range(n_chunks):
                        t = tok_ref[pl.ds(base + LANES * i, LANES)] - lo
                        m = (t >= 0) & (t < W)
                        plsc.store_scatter(sbuf, [row_ids[r8], t], zf, mask=m)
                return c2

            lax.fori_loop(0, n_strips, strip_body, 0)
            if rem_c:
                plo = NT * TILE_C
                for r8 in range(TILE_R):
                    base = r8 * l_pad
                    for i in range(n_chunks):
                        t = tok_ref[pl.ds(base + LANES * i, LANES)] - plo
                        val = vals_ref[pl.ds(base + LANES * i, LANES)]
                        plsc.addupdate_scatter(pbuf, [row_ids[r8], t], val,
                                               mask=t >= 0)
                pltpu.sync_copy(pbuf, out_hbm.at[pl.ds(r0, TILE_R),
                                                 pl.ds(plo, rem_c)])
                for r8 in range(TILE_R):
                    base = r8 * l_pad
                    for i in range(n_chunks):
                        t = tok_ref[pl.ds(base + LANES * i, LANES)] - plo
                        plsc.store_scatter(pbuf, [row_ids[r8], t], zf,
                                           mask=t >= 0)
            return c

        lax.fori_loop(0, rbw, rb_body, 0)

    return run(x_flat, idf)


def kernel(x, idf):
    B, L = x.shape
    return _document_model(x.astype(jnp.int32).reshape(-1), idf, B, L)
